# baseline (speedup 1.0000x reference)
import functools

import jax
import jax.numpy as jnp
from jax import lax
from jax.experimental import pallas as pl
from jax.experimental.pallas import tpu as pltpu
from jax.experimental.pallas import tpu_sc as plsc

D = 64
NC = 2
NS = 16
NW = NC * NS
CHUNK = 128


def _gather_body(n_chunks, idx_hbm, table_hbm, out_hbm,
                 idx_v, buf0, buf1, sem0, sem1):
    wid = lax.axis_index("s") * NC + lax.axis_index("c")
    base = wid * (n_chunks * CHUNK)
    pltpu.sync_copy(idx_hbm.at[wid], idx_v)

    bufs = (buf0, buf1)
    sems = (sem0, sem1)

    pltpu.async_copy(table_hbm.at[idx_v.at[0]], buf0, sem0)

    def body(j, carry):
        def arm(k):
            pltpu.make_async_copy(table_hbm.at[idx_v.at[j]], bufs[k],
                                  sems[k]).wait()

            @pl.when(j + 1 < n_chunks)
            def _():
                pltpu.async_copy(table_hbm.at[idx_v.at[j + 1]],
                                 bufs[1 - k], sems[1 - k])

            pltpu.sync_copy(bufs[k],
                            out_hbm.at[pl.ds(base + j * CHUNK, CHUNK)])

        @pl.when(lax.rem(j, 2) == 0)
        def _():
            arm(0)

        @pl.when(lax.rem(j, 2) == 1)
        def _():
            arm(1)

        return carry

    lax.fori_loop(0, n_chunks, body, 0)


def kernel(x, table):
    b, t = x.shape
    n = b * t
    assert n % (NW * CHUNK) == 0
    n_chunks = n // (NW * CHUNK)

    idx = x.reshape(NW, n_chunks, CHUNK).astype(jnp.int32)

    mesh = plsc.VectorSubcoreMesh(core_axis_name="c", subcore_axis_name="s")
    run = pl.kernel(
        functools.partial(_gather_body, n_chunks),
        out_type=jax.ShapeDtypeStruct((n, D), jnp.float32),
        mesh=mesh,
        scratch_types=[
            pltpu.VMEM((n_chunks, CHUNK), jnp.int32),
            pltpu.VMEM((CHUNK, D), jnp.float32),
            pltpu.VMEM((CHUNK, D), jnp.float32),
            pltpu.SemaphoreType.DMA,
            pltpu.SemaphoreType.DMA,
        ],
        compiler_params=pltpu.CompilerParams(use_tc_tiling_on_sc=False),
    )
    out = run(idx, table)
    return out.reshape(b, t, D)


# speedup vs baseline: 1.0708x; 1.0708x over previous
import functools

import jax
import jax.numpy as jnp
from jax import lax
from jax.experimental import pallas as pl
from jax.experimental.pallas import tpu as pltpu
from jax.experimental.pallas import tpu_sc as plsc

D = 64
NC = 2
NS = 16
NW = NC * NS
CHUNK = 128
G = 4
GROUP = G * CHUNK


def _gather_body(n_chunks, idx_hbm, table_hbm, out_hbm,
                 idx_v, buf_p, buf_q, gsem_p, gsem_q, ssem_p, ssem_q):
    wid = lax.axis_index("s") * NC + lax.axis_index("c")
    base = wid * (n_chunks * CHUNK)
    n_pairs = n_chunks // (2 * G)

    pltpu.sync_copy(idx_hbm.at[wid], idx_v)

    def issue_gathers(buf, sem, g):
        for b in range(G):
            pltpu.async_copy(table_hbm.at[idx_v.at[g * G + b]],
                             buf.at[pl.ds(b * CHUNK, CHUNK)], sem)

    def wait_gathers(buf, sem, g):
        for b in range(G):
            pltpu.make_async_copy(table_hbm.at[idx_v.at[g * G + b]],
                                  buf.at[pl.ds(b * CHUNK, CHUNK)], sem).wait()

    def issue_store(buf, sem, g):
        pltpu.async_copy(buf, out_hbm.at[pl.ds(base + g * GROUP, GROUP)], sem)

    def wait_store(buf, sem, g):
        pltpu.make_async_copy(buf, out_hbm.at[pl.ds(base + g * GROUP, GROUP)],
                              sem).wait()

    issue_gathers(buf_p, gsem_p, 0)

    def outer(i, carry):
        g_p = 2 * i
        g_q = 2 * i + 1

        wait_gathers(buf_p, gsem_p, g_p)
        issue_store(buf_p, ssem_p, g_p)

        @pl.when(i > 0)
        def _():
            wait_store(buf_q, ssem_q, g_q - 2)

        issue_gathers(buf_q, gsem_q, g_q)
        wait_gathers(buf_q, gsem_q, g_q)
        issue_store(buf_q, ssem_q, g_q)
        wait_store(buf_p, ssem_p, g_p)

        @pl.when(i + 1 < n_pairs)
        def _():
            issue_gathers(buf_p, gsem_p, g_p + 2)

        return carry

    lax.fori_loop(0, n_pairs, outer, 0)
    wait_store(buf_q, ssem_q, 2 * n_pairs - 1)


def kernel(x, table):
    b, t = x.shape
    n = b * t
    assert n % (NW * 2 * GROUP) == 0
    n_chunks = n // (NW * CHUNK)

    idx = x.reshape(NW, n_chunks, CHUNK).astype(jnp.int32)

    mesh = plsc.VectorSubcoreMesh(core_axis_name="c", subcore_axis_name="s")
    run = pl.kernel(
        functools.partial(_gather_body, n_chunks),
        out_type=jax.ShapeDtypeStruct((n, D), jnp.float32),
        mesh=mesh,
        scratch_types=[
            pltpu.VMEM((n_chunks, CHUNK), jnp.int32),
            pltpu.VMEM((GROUP, D), jnp.float32),
            pltpu.VMEM((GROUP, D), jnp.float32),
            pltpu.SemaphoreType.DMA,
            pltpu.SemaphoreType.DMA,
            pltpu.SemaphoreType.DMA,
            pltpu.SemaphoreType.DMA,
        ],
        compiler_params=pltpu.CompilerParams(use_tc_tiling_on_sc=False),
    )
    out = run(idx, table)
    return out.reshape(b, t, D)


# speedup vs baseline: 1.0709x; 1.0001x over previous
import functools

import jax
import jax.numpy as jnp
from jax import lax
from jax.experimental import pallas as pl
from jax.experimental.pallas import tpu as pltpu
from jax.experimental.pallas import tpu_sc as plsc

D = 64
NC = 2
NS = 16
NW = NC * NS
CHUNK = 512
G = 1
GROUP = G * CHUNK


def _gather_body(n_chunks, idx_hbm, table_hbm, out_hbm,
                 idx_v, buf_p, buf_q, gsem_p, gsem_q, ssem_p, ssem_q):
    wid = lax.axis_index("s") * NC + lax.axis_index("c")
    base = wid * (n_chunks * CHUNK)
    n_pairs = n_chunks // (2 * G)

    pltpu.sync_copy(idx_hbm.at[wid], idx_v)

    def issue_gathers(buf, sem, g):
        for b in range(G):
            pltpu.async_copy(table_hbm.at[idx_v.at[g * G + b]],
                             buf.at[pl.ds(b * CHUNK, CHUNK)], sem)

    def wait_gathers(buf, sem, g):
        for b in range(G):
            pltpu.make_async_copy(table_hbm.at[idx_v.at[g * G + b]],
                                  buf.at[pl.ds(b * CHUNK, CHUNK)], sem).wait()

    def issue_store(buf, sem, g):
        pltpu.async_copy(buf, out_hbm.at[pl.ds(base + g * GROUP, GROUP)], sem)

    def wait_store(buf, sem, g):
        pltpu.make_async_copy(buf, out_hbm.at[pl.ds(base + g * GROUP, GROUP)],
                              sem).wait()

    issue_gathers(buf_p, gsem_p, 0)

    def outer(i, carry):
        g_p = 2 * i
        g_q = 2 * i + 1

        wait_gathers(buf_p, gsem_p, g_p)
        issue_store(buf_p, ssem_p, g_p)

        @pl.when(i > 0)
        def _():
            wait_store(buf_q, ssem_q, g_q - 2)

        issue_gathers(buf_q, gsem_q, g_q)
        wait_gathers(buf_q, gsem_q, g_q)
        issue_store(buf_q, ssem_q, g_q)
        wait_store(buf_p, ssem_p, g_p)

        @pl.when(i + 1 < n_pairs)
        def _():
            issue_gathers(buf_p, gsem_p, g_p + 2)

        return carry

    lax.fori_loop(0, n_pairs, outer, 0)
    wait_store(buf_q, ssem_q, 2 * n_pairs - 1)


def kernel(x, table):
    b, t = x.shape
    n = b * t
    assert n % (NW * 2 * GROUP) == 0
    n_chunks = n // (NW * CHUNK)

    idx = x.reshape(NW, n_chunks, CHUNK).astype(jnp.int32)

    mesh = plsc.VectorSubcoreMesh(core_axis_name="c", subcore_axis_name="s")
    run = pl.kernel(
        functools.partial(_gather_body, n_chunks),
        out_type=jax.ShapeDtypeStruct((n, D), jnp.float32),
        mesh=mesh,
        scratch_types=[
            pltpu.VMEM((n_chunks, CHUNK), jnp.int32),
            pltpu.VMEM((GROUP, D), jnp.float32),
            pltpu.VMEM((GROUP, D), jnp.float32),
            pltpu.SemaphoreType.DMA,
            pltpu.SemaphoreType.DMA,
            pltpu.SemaphoreType.DMA,
            pltpu.SemaphoreType.DMA,
        ],
        compiler_params=pltpu.CompilerParams(use_tc_tiling_on_sc=False),
    )
    out = run(idx, table)
    return out.reshape(b, t, D)
